# Initial kernel scaffold; baseline (speedup 1.0000x reference)
import jax
import jax.numpy as jnp
from jax.experimental import pallas as pl


def kernel(nodes, to_neighs, u2e, W1, b1, W2, b2, W3, b3):
    raise NotImplementedError("write your pallas kernel here")


# speedup vs baseline: 1.4881x; 1.4881x over previous
import functools

import jax
import jax.numpy as jnp
from jax import lax
from jax.experimental import pallas as pl
from jax.experimental.pallas import tpu as pltpu
from jax.experimental.pallas import tpu_sc as plsc

_NC = 2
_NS = 16
_NW = _NC * _NS

_T = 160


def _sc_gather_body(idx_hbm, nidx_hbm, table_hbm, e_out, u_out,
                    idx_v, nidx_v, rows_v, sem):
    wid = lax.axis_index("s") * _NC + lax.axis_index("c")
    rpw = idx_hbm.shape[0] // _NW
    nrpw = nidx_hbm.shape[0] // _NW
    m = nidx_hbm.shape[1]

    base = wid * rpw
    pltpu.sync_copy(idx_hbm.at[pl.ds(base, rpw)], idx_v)

    def nbody(j, carry):
        pltpu.async_copy(table_hbm.at[idx_v.at[j]], rows_v, sem).wait()
        pltpu.sync_copy(rows_v, e_out.at[pl.ds((base + j) * 128, 128)])
        return carry

    lax.fori_loop(0, rpw, nbody, 0, unroll=False)

    nbase = wid * nrpw
    pltpu.sync_copy(nidx_hbm.at[pl.ds(nbase, nrpw)], nidx_v)
    rows_m = rows_v.at[pl.ds(0, m)]

    def ubody(j, carry):
        pltpu.async_copy(table_hbm.at[nidx_v.at[j]], rows_m, sem).wait()
        pltpu.sync_copy(rows_m, u_out.at[pl.ds((nbase + j) * m, m)])
        return carry

    lax.fori_loop(0, nrpw, ubody, 0, unroll=False)


def _tc_body(e_ref, u_ref, w1a_ref, w1b_ref, b1_ref, w2_ref, b2_ref,
             w3_ref, o_ref):
    t = u_ref.shape[0]
    k = e_ref.shape[0] // t
    d = e_ref.shape[1]
    e = e_ref[...]
    a = jnp.dot(e, w1a_ref[...], preferred_element_type=jnp.float32)
    c = jnp.dot(u_ref[...], w1b_ref[...],
                preferred_element_type=jnp.float32)
    h = a.reshape(t, k, d) + c[:, None, :] + b1_ref[...]
    h = jnp.maximum(h, 0.0).reshape(t * k, d)
    h2 = jnp.dot(h, w2_ref[...], preferred_element_type=jnp.float32)
    h2 = jnp.maximum(h2 + b2_ref[...], 0.0)
    s = jnp.dot(h2, w3_ref[...],
                preferred_element_type=jnp.float32)
    s3 = s.reshape(t, k, 1)
    m = jnp.max(s3, axis=1, keepdims=True)
    w = jnp.exp(s3 - m)
    att = w / jnp.sum(w, axis=1, keepdims=True)
    o_ref[...] = jnp.sum(att * e.reshape(t, k, d), axis=1)


def kernel(nodes, to_neighs, u2e, W1, b1, W2, b2, W3, b3):
    n, k = to_neighs.shape
    v, d = u2e.shape

    npad = ((n + 2 * _T - 1) // (2 * _T)) * (2 * _T)
    m = npad // (8 * _NW)
    assert npad * k % (128 * _NW) == 0 and npad % (8 * _NW) == 0
    assert m % 8 == 0 and m <= 128

    neigh_pad = jnp.zeros((npad, k), jnp.int32).at[:n].set(to_neighs)
    nodes_pad = jnp.zeros((npad,), jnp.int32).at[:n].set(nodes)
    idx2d = neigh_pad.reshape(npad * k // 128, 128)
    nidx2d = nodes_pad.reshape(8 * _NW, m)

    mesh = plsc.VectorSubcoreMesh(core_axis_name="c", subcore_axis_name="s",
                                  num_cores=_NC, num_subcores=_NS)
    gather = pl.kernel(
        _sc_gather_body,
        out_type=(jax.ShapeDtypeStruct((npad * k, d), jnp.float32),
                  jax.ShapeDtypeStruct((npad, d), jnp.float32)),
        mesh=mesh,
        scratch_types=[
            pltpu.VMEM((idx2d.shape[0] // _NW, 128), jnp.int32),
            pltpu.VMEM((nidx2d.shape[0] // _NW, m), jnp.int32),
            pltpu.VMEM((128, d), jnp.float32),
            pltpu.SemaphoreType.DMA,
        ],
    )
    e_u, u_rep = gather(idx2d, nidx2d, u2e)

    grid = npad // _T
    full = lambda i: (0, 0)
    out = pl.pallas_call(
        _tc_body,
        grid=(grid,),
        in_specs=[
            pl.BlockSpec((_T * k, d), lambda i: (i, 0)),
            pl.BlockSpec((_T, d), lambda i: (i, 0)),
            pl.BlockSpec((d, d), full),
            pl.BlockSpec((d, d), full),
            pl.BlockSpec((1, d), full),
            pl.BlockSpec((d, d), full),
            pl.BlockSpec((1, d), full),
            pl.BlockSpec((d, 1), full),
        ],
        out_specs=pl.BlockSpec((_T, d), lambda i: (i, 0)),
        out_shape=jax.ShapeDtypeStruct((npad, d), jnp.float32),
    )(e_u, u_rep, W1[:d], W1[d:], b1.reshape(1, d), W2, b2.reshape(1, d),
      W3)
    return out[:n]


# speedup vs baseline: 1.6435x; 1.1044x over previous
import functools

import jax
import jax.numpy as jnp
from jax import lax
from jax.experimental import pallas as pl
from jax.experimental.pallas import tpu as pltpu
from jax.experimental.pallas import tpu_sc as plsc

_NC = 2
_NS = 16
_NW = _NC * _NS

_T = 160


def _sc_gather_body(idx_hbm, nidx_hbm, table_hbm, e_out, u_out,
                    idx_v, nidx_v, rows_v, nrows_v, gsem, wsem):
    wid = lax.axis_index("s") * _NC + lax.axis_index("c")
    rpw = idx_hbm.shape[0] // _NW
    nrpw = nidx_hbm.shape[0] // _NW
    m = nidx_hbm.shape[1]

    base = wid * rpw
    pltpu.sync_copy(idx_hbm.at[pl.ds(base, rpw)], idx_v)

    def g_start(slot, i):
        pltpu.async_copy(table_hbm.at[idx_v.at[i]], rows_v.at[slot],
                         gsem.at[slot])

    def g_wait(slot):
        pltpu.make_async_copy(table_hbm.at[idx_v.at[0]], rows_v.at[slot],
                              gsem.at[slot]).wait()

    def w_start(slot, i):
        pltpu.async_copy(rows_v.at[slot],
                         e_out.at[pl.ds((base + i) * 128, 128)],
                         wsem.at[slot])

    def w_wait(slot):
        pltpu.make_async_copy(rows_v.at[slot], e_out.at[pl.ds(0, 128)],
                              wsem.at[slot]).wait()

    g_start(0, 0)
    g_start(1, 1)

    def outer(jo2, carry):
        for p in (0, 1):
            jo = jo2 * 2 + p
            a, b = 2 * p, 2 * p + 1
            c, d = 2 * (1 - p), 2 * (1 - p) + 1
            i0 = 2 * jo
            g_wait(a)
            g_wait(b)

            @pl.when(jo > 0)
            def _():
                w_wait(c)
                w_wait(d)

            @pl.when(i0 + 2 < rpw)
            def _():
                g_start(c, i0 + 2)
                g_start(d, i0 + 3)

            w_start(a, i0)
            w_start(b, i0 + 1)
        return carry

    lax.fori_loop(0, rpw // 4, outer, 0, unroll=False)
    w_wait(2)
    w_wait(3)

    nbase = wid * nrpw
    pltpu.sync_copy(nidx_hbm.at[pl.ds(nbase, nrpw)], nidx_v)

    def ubody(j, carry):
        pltpu.async_copy(table_hbm.at[nidx_v.at[j]], nrows_v,
                         gsem.at[0]).wait()
        pltpu.sync_copy(nrows_v, u_out.at[pl.ds((nbase + j) * m, m)])
        return carry

    lax.fori_loop(0, nrpw, ubody, 0, unroll=False)


def _tc_body(e_ref, u_ref, w1a_ref, w1b_ref, b1_ref, w2_ref, b2_ref,
             w3_ref, o_ref):
    t = u_ref.shape[0]
    k = e_ref.shape[0] // t
    d = e_ref.shape[1]
    e = e_ref[...]
    a = jnp.dot(e, w1a_ref[...], preferred_element_type=jnp.float32)
    c = jnp.dot(u_ref[...], w1b_ref[...],
                preferred_element_type=jnp.float32)
    h = a.reshape(t, k, d) + c[:, None, :] + b1_ref[...]
    h = jnp.maximum(h, 0.0).reshape(t * k, d)
    h2 = jnp.dot(h, w2_ref[...], preferred_element_type=jnp.float32)
    h2 = jnp.maximum(h2 + b2_ref[...], 0.0)
    s = jnp.dot(h2, w3_ref[...],
                preferred_element_type=jnp.float32)
    s3 = s.reshape(t, k, 1)
    m = jnp.max(s3, axis=1, keepdims=True)
    w = jnp.exp(s3 - m)
    att = w / jnp.sum(w, axis=1, keepdims=True)
    o_ref[...] = jnp.sum(att * e.reshape(t, k, d), axis=1)


def kernel(nodes, to_neighs, u2e, W1, b1, W2, b2, W3, b3):
    n, k = to_neighs.shape
    v, d = u2e.shape

    npad = ((n + 2 * _T - 1) // (2 * _T)) * (2 * _T)
    m = npad // (8 * _NW)
    assert npad * k % (128 * _NW) == 0 and npad % (8 * _NW) == 0
    assert m % 8 == 0 and m <= 128

    neigh_pad = jnp.zeros((npad, k), jnp.int32).at[:n].set(to_neighs)
    nodes_pad = jnp.zeros((npad,), jnp.int32).at[:n].set(nodes)
    idx2d = neigh_pad.reshape(npad * k // 128, 128)
    nidx2d = nodes_pad.reshape(8 * _NW, m)

    mesh = plsc.VectorSubcoreMesh(core_axis_name="c", subcore_axis_name="s",
                                  num_cores=_NC, num_subcores=_NS)
    gather = pl.kernel(
        _sc_gather_body,
        out_type=(jax.ShapeDtypeStruct((npad * k, d), jnp.float32),
                  jax.ShapeDtypeStruct((npad, d), jnp.float32)),
        mesh=mesh,
        scratch_types=[
            pltpu.VMEM((idx2d.shape[0] // _NW, 128), jnp.int32),
            pltpu.VMEM((nidx2d.shape[0] // _NW, m), jnp.int32),
            pltpu.VMEM((4, 128, d), jnp.float32),
            pltpu.VMEM((m, d), jnp.float32),
            pltpu.SemaphoreType.DMA((4,)),
            pltpu.SemaphoreType.DMA((4,)),
        ],
    )
    e_u, u_rep = gather(idx2d, nidx2d, u2e)

    grid = npad // _T
    full = lambda i: (0, 0)
    out = pl.pallas_call(
        _tc_body,
        grid=(grid,),
        in_specs=[
            pl.BlockSpec((_T * k, d), lambda i: (i, 0)),
            pl.BlockSpec((_T, d), lambda i: (i, 0)),
            pl.BlockSpec((d, d), full),
            pl.BlockSpec((d, d), full),
            pl.BlockSpec((1, d), full),
            pl.BlockSpec((d, d), full),
            pl.BlockSpec((1, d), full),
            pl.BlockSpec((d, 1), full),
        ],
        out_specs=pl.BlockSpec((_T, d), lambda i: (i, 0)),
        out_shape=jax.ShapeDtypeStruct((npad, d), jnp.float32),
    )(e_u, u_rep, W1[:d], W1[d:], b1.reshape(1, d), W2, b2.reshape(1, d),
      W3)
    return out[:n]
